# Initial kernel scaffold; baseline (speedup 1.0000x reference)
import jax
import jax.numpy as jnp
from jax.experimental import pallas as pl


def kernel(input):
    raise NotImplementedError("write your pallas kernel here")


# speedup vs baseline: 2.3961x; 2.3961x over previous
import jax
import jax.numpy as jnp
import numpy as np
from jax import lax
from jax.experimental import pallas as pl
from jax.experimental.pallas import tpu as pltpu
from jax.experimental.pallas import tpu_sc as plsc

N = 1_200_000
NW = 32
W = 37_504
C = 4_688
NCHUNK = W // C
G = C // 16

_VS = np.array([0.05, 0.05, 0.1], dtype=np.float32)
RX, RY, RZ = 0.0, -40.0, -3.0
IVX = float(1.0 / np.float64(_VS[0]))
IVY = float(1.0 / np.float64(_VS[1]))
IVZ = float(1.0 / np.float64(_VS[2]))
GX, GY, GZ = 1408, 1600, 40


def _sc_body(pts_hbm, out_hbm,
             p0, p1, z0, z1, y0, y1, x0, x1,
             si0, si1, so0, so1):
    cid = lax.axis_index("c")
    sid = lax.axis_index("s")
    wid = sid * 2 + cid
    base = jnp.minimum(wid * W, N - W)
    base = pl.multiple_of(base, 16)

    pbufs = (p0, p1)
    zbufs = (z0, z1)
    ybufs = (y0, y1)
    xbufs = (x0, x1)
    isems = (si0, si1)
    osems = (so0, so1)

    iota = lax.iota(jnp.int32, 16)
    ix0 = iota * 4
    iy0 = ix0 + 1
    iz0 = ix0 + 2

    def compute(pb, zb, yb, xb):
        def body(g, carry):
            off = g * 64
            vx = plsc.load_gather(pb, [ix0 + off])
            vy = plsc.load_gather(pb, [iy0 + off])
            vz = plsc.load_gather(pb, [iz0 + off])
            tx = (vx - RX) * IVX
            ty = (vy - RY) * IVY
            tz = (vz - RZ) * IVZ
            cx = tx.astype(jnp.int32)
            cy = ty.astype(jnp.int32)
            cz = tz.astype(jnp.int32)
            ok = ((tx >= 0.0) & (cx < GX)
                  & (ty >= 0.0) & (cy < GY)
                  & (tz >= 0.0) & (cz < GZ))
            s = g * 16
            zb[pl.ds(s, 16)] = jnp.where(ok, cz, -1)
            yb[pl.ds(s, 16)] = jnp.where(ok, cy, -1)
            xb[pl.ds(s, 16)] = jnp.where(ok, cx, -1)
            return carry
        lax.fori_loop(0, G, body, 0)

    in_d = [None, None]
    out_d = [[], []]
    in_d[0] = pltpu.async_copy(
        pts_hbm.at[pl.ds(base * 4, C * 4)], pbufs[0], isems[0])
    for k in range(NCHUNK):
        b = k & 1
        for d in out_d[b]:
            d.wait()
        out_d[b] = []
        in_d[b].wait()
        if k + 1 < NCHUNK:
            nb = 1 - b
            noff = (base + (k + 1) * C) * 4
            in_d[nb] = pltpu.async_copy(
                pts_hbm.at[pl.ds(noff, C * 4)], pbufs[nb], isems[nb])
        compute(pbufs[b], zbufs[b], ybufs[b], xbufs[b])
        ob = base + k * C
        out_d[b].append(pltpu.async_copy(
            zbufs[b], out_hbm.at[pl.ds(ob, C)], osems[b]))
        out_d[b].append(pltpu.async_copy(
            ybufs[b], out_hbm.at[pl.ds(N + ob, C)], osems[b]))
        out_d[b].append(pltpu.async_copy(
            xbufs[b], out_hbm.at[pl.ds(2 * N + ob, C)], osems[b]))
    for b in (0, 1):
        for d in out_d[b]:
            d.wait()


_sc_call = pl.kernel(
    _sc_body,
    out_type=jax.ShapeDtypeStruct((3 * N,), jnp.int32),
    mesh=plsc.VectorSubcoreMesh(core_axis_name="c", subcore_axis_name="s"),
    compiler_params=pltpu.CompilerParams(needs_layout_passes=False),
    scratch_types=[
        pltpu.VMEM((C * 4,), jnp.float32),
        pltpu.VMEM((C * 4,), jnp.float32),
        pltpu.VMEM((C,), jnp.int32),
        pltpu.VMEM((C,), jnp.int32),
        pltpu.VMEM((C,), jnp.int32),
        pltpu.VMEM((C,), jnp.int32),
        pltpu.VMEM((C,), jnp.int32),
        pltpu.VMEM((C,), jnp.int32),
        pltpu.SemaphoreType.DMA,
        pltpu.SemaphoreType.DMA,
        pltpu.SemaphoreType.DMA,
        pltpu.SemaphoreType.DMA,
    ],
)


def kernel(input):
    flat = input.reshape(-1)
    out = _sc_call(flat)
    return out.reshape(3, N)
